# Initial kernel scaffold; baseline (speedup 1.0000x reference)
import jax
import jax.numpy as jnp
from jax.experimental import pallas as pl


def kernel(emb, indices, small_dists):
    raise NotImplementedError("write your pallas kernel here")


# speedup vs baseline: 2.5200x; 2.5200x over previous
import functools

import jax
import jax.numpy as jnp
from jax import lax
from jax.experimental import pallas as pl
from jax.experimental.pallas import tpu as pltpu
from jax.experimental.pallas import tpu_sc as plsc

N = 16384
D = 64
L = 16
NC = 2
NS = 16
NW = NC * NS
B = 128


def _rsqrt_nr(x):
    xh = x * 0.5
    i = plsc.bitcast(x, jnp.int32)
    i = jnp.int32(0x5F3759DF) - (i >> 1)
    y = plsc.bitcast(i, jnp.float32)
    y = y * (1.5 - xh * y * y)
    y = y * (1.5 - xh * y * y)
    y = y * (1.5 - xh * y * y)
    return y


def _make_sc_kernel(nblk):
    mesh = plsc.VectorSubcoreMesh(core_axis_name="c", subcore_axis_name="s")

    @functools.partial(
        pl.kernel,
        mesh=mesh,
        compiler_params=pltpu.CompilerParams(
            needs_layout_passes=False, use_tc_tiling_on_sc=False),
        out_type=jax.ShapeDtypeStruct((NW, L), jnp.float32),
        scratch_types=[
            pltpu.VMEM((B,), jnp.int32),
            pltpu.VMEM((B,), jnp.int32),
            pltpu.VMEM((B,), jnp.float32),
            pltpu.VMEM((B, D), jnp.float32),
            pltpu.VMEM((B, D), jnp.float32),
            pltpu.VMEM((L,), jnp.float32),
            pltpu.SemaphoreType.DMA,
        ],
    )
    def sc_kernel(emb_hbm, ii_hbm, jj_hbm, sd_hbm, out_hbm,
                  ii_v, jj_v, sd_v, ri_v, rj_v, acc_v, sem):
        wid = lax.axis_index("s") * NC + lax.axis_index("c")
        lane = lax.broadcasted_iota(jnp.int32, (L,), 0)

        def block(t, acc):
            base = (wid * nblk + t) * B
            pltpu.sync_copy(ii_hbm.at[pl.ds(base, B)], ii_v)
            pltpu.sync_copy(jj_hbm.at[pl.ds(base, B)], jj_v)
            pltpu.sync_copy(sd_hbm.at[pl.ds(base, B)], sd_v)
            pltpu.async_copy(emb_hbm.at[ii_v], ri_v, sem).wait()
            pltpu.async_copy(emb_hbm.at[jj_v], rj_v, sem).wait()
            for g in range(B // L):
                tot = jnp.zeros((L,), jnp.float32)
                for r in range(L):
                    p = g * L + r
                    s = jnp.zeros((L,), jnp.float32)
                    for k in range(D // L):
                        vi = ri_v[p, pl.ds(k * L, L)]
                        vj = rj_v[p, pl.ds(k * L, L)]
                        df = vi - vj
                        s = s + df * df
                    tot = jnp.where(lane == r, jnp.sum(s), tot)
                dist = tot * _rsqrt_nr(jnp.maximum(tot, 1e-30))
                res = sd_v[pl.ds(g * L, L)] - dist
                acc = acc + res * res
            return acc

        acc = lax.fori_loop(0, nblk, block, jnp.zeros((L,), jnp.float32))
        acc_v[...] = acc
        pltpu.sync_copy(acc_v, out_hbm.at[wid])

    return sc_kernel


def kernel(emb, indices, small_dists):
    nnz = indices.shape[0]
    nblk = -(-nnz // (NW * B))
    pad = NW * nblk * B - nnz
    ii = jnp.pad(indices[:, 0], (0, pad))
    jj = jnp.pad(indices[:, 1], (0, pad))
    sd = jnp.pad(small_dists, (0, pad))
    partial = _make_sc_kernel(nblk)(emb, ii, jj, sd)
    return jnp.sum(partial)
